# Initial kernel scaffold; baseline (speedup 1.0000x reference)
import jax
import jax.numpy as jnp
from jax.experimental import pallas as pl


def kernel(h, x, edge_index, We1, be1, We2, be2, Wn1, bn1, Wn2, bn2, Wc1, bc1, Wc2):
    raise NotImplementedError("write your pallas kernel here")


# speedup vs baseline: 2.2724x; 2.2724x over previous
import functools

import jax
import jax.numpy as jnp
from jax import lax
from jax.experimental import pallas as pl
from jax.experimental.pallas import tpu as pltpu
from jax.experimental.pallas import tpu_sc as plsc

N = 50000
E = 800000
D = 64

NPAD = 51200
EPAD = 819200
NW = 32
EW = EPAD // NW
GC = 512
GCH = EW // 1024
ET = EPAD // 16
SCH = ET // 1024

_mesh = plsc.VectorSubcoreMesh(core_axis_name="c", subcore_axis_name="s")


@functools.partial(
    pl.kernel,
    mesh=_mesh,
    compiler_params=pltpu.CompilerParams(use_tc_tiling_on_sc=False),
    out_type=[
        jax.ShapeDtypeStruct((EPAD, D), jnp.float32),
        jax.ShapeDtypeStruct((EPAD, 16), jnp.float32),
    ],
    scratch_types=[
        pltpu.VMEM((8, 128), jnp.int32),
        pltpu.VMEM((8, 128), jnp.int32),
        pltpu.VMEM((GC, D), jnp.float32),
        pltpu.VMEM((GC, D), jnp.float32),
        pltpu.VMEM((GC, 16), jnp.float32),
        pltpu.VMEM((GC, 16), jnp.float32),
        pltpu.SemaphoreType.DMA,
    ],
)
def _gather_sc(a_hbm, b_hbm, x_hbm, row_hbm, col_hbm, s_out, rel_out,
               rowv, colv, bufa, bufb, bufxr, bufxc, sem):
    wid = lax.axis_index("s") * 2 + lax.axis_index("c")
    base = wid * EW

    def chunk(g, _):
        mb = base + g * 1024
        mrow = pl.multiple_of(mb // 128, 8)
        pltpu.sync_copy(row_hbm.at[pl.ds(mrow, 8)], rowv)
        pltpu.sync_copy(col_hbm.at[pl.ds(mrow, 8)], colv)
        for hc in range(2):
            eb = pl.multiple_of(mb + hc * GC, 8)
            handles = []
            for j in range(GC // 128):
                dst = pl.ds(j * 128, 128)
                k = hc * 4 + j
                handles.append(pltpu.async_copy(a_hbm.at[rowv.at[k]], bufa.at[dst], sem))
                handles.append(pltpu.async_copy(b_hbm.at[colv.at[k]], bufb.at[dst], sem))
                handles.append(pltpu.async_copy(x_hbm.at[rowv.at[k]], bufxr.at[dst], sem))
                handles.append(pltpu.async_copy(x_hbm.at[colv.at[k]], bufxc.at[dst], sem))
            for h in handles:
                h.wait()

            def vrow(i, _):
                r = i * 4
                for rr in range(4):
                    for j in range(D // 16):
                        c = pl.ds(j * 16, 16)
                        bufa[r + rr, c] = bufa[r + rr, c] + bufb[r + rr, c]
                    c16 = pl.ds(0, 16)
                    bufxr[r + rr, c16] = bufxr[r + rr, c16] - bufxc[r + rr, c16]
                return 0

            lax.fori_loop(0, GC // 4, vrow, 0)
            pltpu.sync_copy(bufa, s_out.at[pl.ds(eb, GC)])
            pltpu.sync_copy(bufxr, rel_out.at[pl.ds(eb, GC)])
        return 0

    lax.fori_loop(0, GCH, chunk, 0)


QTR = 12500
RSH = 12544
SSUB = 512


@functools.partial(
    pl.kernel,
    mesh=_mesh,
    compiler_params=pltpu.CompilerParams(use_tc_tiling_on_sc=False),
    out_type=[
        jax.ShapeDtypeStruct((N, D), jnp.float32),
        jax.ShapeDtypeStruct((N, 16), jnp.float32),
    ],
    scratch_types=[
        pltpu.VMEM((8, 128), jnp.int32),
        pltpu.VMEM((4, 128), jnp.int32),
        pltpu.VMEM((SSUB, D), jnp.float32),
        pltpu.VMEM((SSUB, 16), jnp.float32),
        pltpu.VMEM_SHARED((RSH, D), jnp.float32),
        pltpu.VMEM_SHARED((RSH, 16), jnp.float32),
    ],
)
def _scatter_sc(msg_hbm, cd_hbm, col_hbm, agg_out, xacc_out,
                colv, idxv, msgbuf, cdbuf, agg_sh, xacc_sh):
    c = lax.axis_index("c")
    s = lax.axis_index("s")

    for p in range(2):
        nbase = (2 * p + c) * QTR

        def zrow(i, _):
            for j in range(D // 16):
                msgbuf[i, pl.ds(j * 16, 16)] = jnp.zeros((16,), jnp.float32)
            cdbuf[i, pl.ds(0, 16)] = jnp.zeros((16,), jnp.float32)
            return 0

        lax.fori_loop(0, SSUB, zrow, 0)
        r0 = pl.multiple_of(s * (RSH // 16), 8)
        pltpu.sync_copy(msgbuf, agg_sh.at[pl.ds(r0, SSUB)])
        pltpu.sync_copy(msgbuf.at[pl.ds(0, 272)], agg_sh.at[pl.ds(r0 + SSUB, 272)])
        pltpu.sync_copy(cdbuf, xacc_sh.at[pl.ds(r0, SSUB)])
        pltpu.sync_copy(cdbuf.at[pl.ds(0, 272)], xacc_sh.at[pl.ds(r0 + SSUB, 272)])
        plsc.subcore_barrier()

        def chunk(g, _):
            mb = s * ET + g * 1024
            mrow = pl.multiple_of(mb // 128, 8)
            pltpu.sync_copy(col_hbm.at[pl.ds(mrow, 8)], colv)
            for hc in range(2):
                eb = pl.multiple_of(mb + hc * SSUB, 8)

                def ibody(j, _):
                    for i in range(8):
                        v = colv[hc * 4 + j, pl.ds(i * 16, 16)]
                        l = v - nbase
                        ok = (l >= 0) & (l < QTR)
                        idxv[j, pl.ds(i * 16, 16)] = jnp.where(ok, l, QTR)
                    return 0

                lax.fori_loop(0, 4, ibody, 0)
                pltpu.sync_copy(msg_hbm.at[pl.ds(eb, SSUB)], msgbuf)
                pltpu.sync_copy(cd_hbm.at[pl.ds(eb, SSUB)], cdbuf)
                for j in range(4):
                    src = pl.ds(j * 128, 128)
                    pltpu.sync_copy(msgbuf.at[src], agg_sh.at[idxv.at[j]], add=True)
                    pltpu.sync_copy(cdbuf.at[src], xacc_sh.at[idxv.at[j]], add=True)
            return 0

        lax.fori_loop(0, SCH, chunk, 0)
        plsc.subcore_barrier()

        ob = s * 780
        og = nbase + s * 780
        pltpu.sync_copy(agg_sh.at[pl.ds(ob, 780)], agg_out.at[pl.ds(og, 780)])
        pltpu.sync_copy(xacc_sh.at[pl.ds(ob, 780)], xacc_out.at[pl.ds(og, 780)])

        @pl.when(s == 0)
        def _():
            pltpu.sync_copy(agg_sh.at[pl.ds(12480, 20)], agg_out.at[pl.ds(nbase + 12480, 20)])
            pltpu.sync_copy(xacc_sh.at[pl.ds(12480, 20)], xacc_out.at[pl.ds(nbase + 12480, 20)])

        plsc.subcore_barrier()


def _silu(v):
    return v * jax.nn.sigmoid(v)


def _pre_body(h_ref, w1r_ref, w1c_ref, a_ref, b_ref):
    h = h_ref[...]
    a_ref[...] = jnp.dot(h, w1r_ref[...], preferred_element_type=jnp.float32)
    b_ref[...] = jnp.dot(h, w1c_ref[...], preferred_element_type=jnp.float32)


def _edge_body(s_ref, rel_ref, w1d_ref, be1_ref, we2_ref, be2_ref,
               wc1_ref, bc1_ref, wc2_ref, msg_ref, cd_ref):
    rel = rel_ref[...]
    dist = jnp.sqrt(jnp.sum(rel * rel, axis=-1, keepdims=True))
    pre = s_ref[...] + dist * w1d_ref[...] + be1_ref[...]
    t = _silu(pre)
    msg = jnp.dot(t, we2_ref[...], preferred_element_type=jnp.float32) + be2_ref[...]
    cpre = jnp.dot(msg, wc1_ref[...], preferred_element_type=jnp.float32) + bc1_ref[...]
    cw = jnp.sum(_silu(cpre) * wc2_ref[...], axis=-1, keepdims=True)
    msg_ref[...] = msg
    cd_ref[...] = rel * cw


def _node_body(h_ref, agg_ref, x_ref, xacc_ref, wn1a_ref, wn1b_ref, bn1_ref,
               wn2_ref, bn2_ref, hn_ref, xn_ref):
    h = h_ref[...]
    pre = (jnp.dot(h, wn1a_ref[...], preferred_element_type=jnp.float32)
           + jnp.dot(agg_ref[...], wn1b_ref[...], preferred_element_type=jnp.float32)
           + bn1_ref[...])
    hn_ref[...] = h + jnp.dot(_silu(pre), wn2_ref[...],
                              preferred_element_type=jnp.float32) + bn2_ref[...]
    xn_ref[...] = x_ref[...] + xacc_ref[...]


def _full(shape):
    return pl.BlockSpec(shape, lambda i: (0, 0))


def kernel(h, x, edge_index, We1, be1, We2, be2, Wn1, bn1, Wn2, bn2, Wc1, bc1, Wc2):
    f32 = jnp.float32
    row = edge_index[0]
    col = edge_index[1]
    pad = jnp.full((EPAD - E,), N, dtype=jnp.int32)
    row_p = jnp.concatenate([row, pad]).reshape(EPAD // 128, 128)
    col_p = jnp.concatenate([col, pad]).reshape(EPAD // 128, 128)

    h_pad = jnp.zeros((NPAD, D), f32).at[:N].set(h)
    x_pad = jnp.zeros((NPAD, 16), f32).at[:N, :3].set(x)
    x16 = jnp.zeros((N, 16), f32).at[:, :3].set(x)

    bn = 2048
    a_tab, b_tab = pl.pallas_call(
        _pre_body,
        grid=(NPAD // bn,),
        in_specs=[pl.BlockSpec((bn, D), lambda i: (i, 0)), _full((D, D)), _full((D, D))],
        out_specs=[pl.BlockSpec((bn, D), lambda i: (i, 0))] * 2,
        out_shape=[jax.ShapeDtypeStruct((NPAD, D), f32)] * 2,
    )(h_pad, We1[:D], We1[D:2 * D])

    s_sum, rel = _gather_sc(a_tab, b_tab, x_pad, row_p, col_p)

    be = 2048
    w1d = We1[2 * D:2 * D + 1]
    msg, cd = pl.pallas_call(
        _edge_body,
        grid=(EPAD // be,),
        in_specs=[pl.BlockSpec((be, D), lambda i: (i, 0)),
                  pl.BlockSpec((be, 16), lambda i: (i, 0)),
                  _full((1, D)), _full((1, D)), _full((D, D)), _full((1, D)),
                  _full((D, D)), _full((1, D)), _full((1, D))],
        out_specs=[pl.BlockSpec((be, D), lambda i: (i, 0)),
                   pl.BlockSpec((be, 16), lambda i: (i, 0))],
        out_shape=[jax.ShapeDtypeStruct((EPAD, D), f32),
                   jax.ShapeDtypeStruct((EPAD, 16), f32)],
    )(s_sum, rel, w1d, be1.reshape(1, D), We2, be2.reshape(1, D),
      Wc1, bc1.reshape(1, D), Wc2.reshape(1, D))

    agg, xacc = _scatter_sc(msg, cd, col_p)

    bnn = 2000
    h_new, x_new16 = pl.pallas_call(
        _node_body,
        grid=(N // bnn,),
        in_specs=[pl.BlockSpec((bnn, D), lambda i: (i, 0)),
                  pl.BlockSpec((bnn, D), lambda i: (i, 0)),
                  pl.BlockSpec((bnn, 16), lambda i: (i, 0)),
                  pl.BlockSpec((bnn, 16), lambda i: (i, 0)),
                  _full((D, D)), _full((D, D)), _full((1, D)),
                  _full((D, D)), _full((1, D))],
        out_specs=[pl.BlockSpec((bnn, D), lambda i: (i, 0)),
                   pl.BlockSpec((bnn, 16), lambda i: (i, 0))],
        out_shape=[jax.ShapeDtypeStruct((N, D), f32),
                   jax.ShapeDtypeStruct((N, 16), f32)],
    )(h, agg, x16, xacc, Wn1[:D], Wn1[D:], bn1.reshape(1, D), Wn2, bn2.reshape(1, D))

    return h_new, x_new16[:, :3]


# speedup vs baseline: 2.2910x; 1.0082x over previous
import functools

import jax
import jax.numpy as jnp
from jax import lax
from jax.experimental import pallas as pl
from jax.experimental.pallas import tpu as pltpu
from jax.experimental.pallas import tpu_sc as plsc

N = 50000
E = 800000
D = 64
W = 80

NPAD = 51200
EPAD = 819200
NW = 32
EW = EPAD // NW
GC = 512
GCH = EW // GC
ET = EPAD // 16
SSUB = 640
SCH = ET // SSUB
QTR = 12500
RSH = 12544

_mesh = plsc.VectorSubcoreMesh(core_axis_name="c", subcore_axis_name="s")


@functools.partial(
    pl.kernel,
    mesh=_mesh,
    compiler_params=pltpu.CompilerParams(use_tc_tiling_on_sc=False),
    out_type=jax.ShapeDtypeStruct((EPAD, W), jnp.float32),
    scratch_types=[
        pltpu.VMEM((GC,), jnp.int32),
        pltpu.VMEM((GC,), jnp.int32),
        pltpu.VMEM((GC, W), jnp.float32),
        pltpu.VMEM((GC, W), jnp.float32),
        pltpu.SemaphoreType.DMA,
    ],
)
def _gather_sc(ta_hbm, tb_hbm, row_hbm, col_hbm, sr_out, rowv, colv, bufa, bufb, sem):
    wid = lax.axis_index("s") * 2 + lax.axis_index("c")
    base = wid * EW

    def chunk(g, _):
        eb = pl.multiple_of(base + g * GC, 8)
        pltpu.sync_copy(row_hbm.at[pl.ds(eb, GC)], rowv)
        pltpu.sync_copy(col_hbm.at[pl.ds(eb, GC)], colv)
        h1 = pltpu.async_copy(ta_hbm.at[rowv], bufa, sem)
        h2 = pltpu.async_copy(tb_hbm.at[colv], bufb, sem)
        h1.wait()
        h2.wait()

        def vrow(i, _):
            r = i * 4
            for rr in range(4):
                for j in range(4):
                    cs = pl.ds(j * 16, 16)
                    bufa[r + rr, cs] = bufa[r + rr, cs] + bufb[r + rr, cs]
                cs = pl.ds(64, 16)
                bufa[r + rr, cs] = bufa[r + rr, cs] - bufb[r + rr, cs]
            return 0

        lax.fori_loop(0, GC // 4, vrow, 0)
        pltpu.sync_copy(bufa, sr_out.at[pl.ds(eb, GC)])
        return 0

    lax.fori_loop(0, GCH, chunk, 0)


@functools.partial(
    pl.kernel,
    mesh=_mesh,
    compiler_params=pltpu.CompilerParams(use_tc_tiling_on_sc=False),
    out_type=jax.ShapeDtypeStruct((N, W), jnp.float32),
    scratch_types=[
        pltpu.VMEM((SSUB,), jnp.int32),
        pltpu.VMEM((SSUB,), jnp.int32),
        pltpu.VMEM((SSUB, W), jnp.float32),
        pltpu.VMEM_SHARED((RSH, W), jnp.float32),
    ],
)
def _scatter_sc(mc_hbm, col_hbm, acc_out, colv, idxv, databuf, acc_sh):
    c = lax.axis_index("c")
    s = lax.axis_index("s")

    for p in range(2):
        nbase = (2 * p + c) * QTR

        def zrow(i, _):
            for j in range(W // 16):
                databuf[i, pl.ds(j * 16, 16)] = jnp.zeros((16,), jnp.float32)
            return 0

        lax.fori_loop(0, SSUB, zrow, 0)
        r0 = pl.multiple_of(s * (RSH // 16), 8)
        pltpu.sync_copy(databuf, acc_sh.at[pl.ds(r0, SSUB)])
        pltpu.sync_copy(databuf.at[pl.ds(0, 144)], acc_sh.at[pl.ds(r0 + SSUB, 144)])
        plsc.subcore_barrier()

        def chunk(g, _):
            eb = pl.multiple_of(s * ET + g * SSUB, 8)
            pltpu.sync_copy(col_hbm.at[pl.ds(eb, SSUB)], colv)

            def ibody(i, _):
                v = colv[pl.ds(i * 16, 16)]
                l = v - nbase
                ok = (l >= 0) & (l < QTR)
                idxv[pl.ds(i * 16, 16)] = jnp.where(ok, l, QTR)
                return 0

            lax.fori_loop(0, SSUB // 16, ibody, 0)
            pltpu.sync_copy(mc_hbm.at[pl.ds(eb, SSUB)], databuf)
            pltpu.sync_copy(databuf, acc_sh.at[idxv], add=True)
            return 0

        lax.fori_loop(0, SCH, chunk, 0)
        plsc.subcore_barrier()

        ob = s * 780
        og = nbase + s * 780
        pltpu.sync_copy(acc_sh.at[pl.ds(ob, 780)], acc_out.at[pl.ds(og, 780)])

        @pl.when(s == 0)
        def _():
            pltpu.sync_copy(acc_sh.at[pl.ds(12480, 20)], acc_out.at[pl.ds(nbase + 12480, 20)])

        plsc.subcore_barrier()


def _silu(v):
    return v * jax.nn.sigmoid(v)


def _pre_body(h_ref, x_ref, w1r_ref, w1c_ref, ta_ref, tb_ref):
    h = h_ref[...]
    x16 = x_ref[...]
    ta_ref[...] = jnp.concatenate(
        [jnp.dot(h, w1r_ref[...], preferred_element_type=jnp.float32), x16], axis=1)
    tb_ref[...] = jnp.concatenate(
        [jnp.dot(h, w1c_ref[...], preferred_element_type=jnp.float32), x16], axis=1)


def _edge_body(sr_ref, w1d_ref, be1_ref, we2_ref, be2_ref,
               wc1_ref, bc1_ref, wc2_ref, mc_ref):
    sr = sr_ref[...]
    rel = sr[:, D:W]
    dist = jnp.sqrt(jnp.sum(rel * rel, axis=-1, keepdims=True))
    pre = sr[:, :D] + dist * w1d_ref[...] + be1_ref[...]
    t = _silu(pre)
    msg = jnp.dot(t, we2_ref[...], preferred_element_type=jnp.float32) + be2_ref[...]
    cpre = jnp.dot(msg, wc1_ref[...], preferred_element_type=jnp.float32) + bc1_ref[...]
    cw = jnp.sum(_silu(cpre) * wc2_ref[...], axis=-1, keepdims=True)
    mc_ref[...] = jnp.concatenate([msg, rel * cw], axis=1)


def _node_body(h_ref, acc_ref, x_ref, wn1a_ref, wn1b_ref, bn1_ref,
               wn2_ref, bn2_ref, hn_ref, xn_ref):
    h = h_ref[...]
    acc = acc_ref[...]
    pre = (jnp.dot(h, wn1a_ref[...], preferred_element_type=jnp.float32)
           + jnp.dot(acc[:, :D], wn1b_ref[...], preferred_element_type=jnp.float32)
           + bn1_ref[...])
    hn_ref[...] = h + jnp.dot(_silu(pre), wn2_ref[...],
                              preferred_element_type=jnp.float32) + bn2_ref[...]
    xn_ref[...] = x_ref[...] + acc[:, D:W]


def _full(shape):
    return pl.BlockSpec(shape, lambda i: (0, 0))


def kernel(h, x, edge_index, We1, be1, We2, be2, Wn1, bn1, Wn2, bn2, Wc1, bc1, Wc2):
    f32 = jnp.float32
    row = edge_index[0]
    col = edge_index[1]
    pad = jnp.full((EPAD - E,), N, dtype=jnp.int32)
    row_p = jnp.concatenate([row, pad])
    col_p = jnp.concatenate([col, pad])

    h_pad = jnp.zeros((NPAD, D), f32).at[:N].set(h)
    x_pad = jnp.zeros((NPAD, 16), f32).at[:N, :3].set(x)
    x16 = x_pad[:N]

    bn = 2048
    ta, tb = pl.pallas_call(
        _pre_body,
        grid=(NPAD // bn,),
        in_specs=[pl.BlockSpec((bn, D), lambda i: (i, 0)),
                  pl.BlockSpec((bn, 16), lambda i: (i, 0)),
                  _full((D, D)), _full((D, D))],
        out_specs=[pl.BlockSpec((bn, W), lambda i: (i, 0))] * 2,
        out_shape=[jax.ShapeDtypeStruct((NPAD, W), f32)] * 2,
    )(h_pad, x_pad, We1[:D], We1[D:2 * D])

    sr = _gather_sc(ta, tb, row_p, col_p)

    be = 2048
    w1d = We1[2 * D:2 * D + 1]
    mc = pl.pallas_call(
        _edge_body,
        grid=(EPAD // be,),
        in_specs=[pl.BlockSpec((be, W), lambda i: (i, 0)),
                  _full((1, D)), _full((1, D)), _full((D, D)), _full((1, D)),
                  _full((D, D)), _full((1, D)), _full((1, D))],
        out_specs=pl.BlockSpec((be, W), lambda i: (i, 0)),
        out_shape=jax.ShapeDtypeStruct((EPAD, W), f32),
    )(sr, w1d, be1.reshape(1, D), We2, be2.reshape(1, D),
      Wc1, bc1.reshape(1, D), Wc2.reshape(1, D))

    acc = _scatter_sc(mc, col_p)

    bnn = 2000
    h_new, x_new16 = pl.pallas_call(
        _node_body,
        grid=(N // bnn,),
        in_specs=[pl.BlockSpec((bnn, D), lambda i: (i, 0)),
                  pl.BlockSpec((bnn, W), lambda i: (i, 0)),
                  pl.BlockSpec((bnn, 16), lambda i: (i, 0)),
                  _full((D, D)), _full((D, D)), _full((1, D)),
                  _full((D, D)), _full((1, D))],
        out_specs=[pl.BlockSpec((bnn, D), lambda i: (i, 0)),
                   pl.BlockSpec((bnn, 16), lambda i: (i, 0))],
        out_shape=[jax.ShapeDtypeStruct((N, D), f32),
                   jax.ShapeDtypeStruct((N, 16), f32)],
    )(h, acc, x16, Wn1[:D], Wn1[D:], bn1.reshape(1, D), Wn2, bn2.reshape(1, D))

    return h_new, x_new16[:, :3]


# speedup vs baseline: 2.3972x; 1.0464x over previous
import functools

import jax
import jax.numpy as jnp
from jax import lax
from jax.experimental import pallas as pl
from jax.experimental.pallas import tpu as pltpu
from jax.experimental.pallas import tpu_sc as plsc

N = 50000
E = 800000
D = 64
W = 128

NPAD = 51200
EPAD = 819200
NW = 32
EW = EPAD // NW
GC = 512
GCH = EW // GC
ET = EPAD // 16
SSUB = 512
SCH = ET // SSUB
QTR = 12500
RSH = 12544

_mesh = plsc.VectorSubcoreMesh(core_axis_name="c", subcore_axis_name="s")
_bf16 = jnp.bfloat16


@functools.partial(
    pl.kernel,
    mesh=_mesh,
    compiler_params=pltpu.CompilerParams(use_tc_tiling_on_sc=False),
    out_type=jax.ShapeDtypeStruct((EPAD, W), _bf16),
    scratch_types=[
        pltpu.VMEM((GC,), jnp.int32),
        pltpu.VMEM((GC,), jnp.int32),
        pltpu.VMEM((GC, W), _bf16),
        pltpu.VMEM((GC, W), _bf16),
        pltpu.SemaphoreType.DMA,
    ],
)
def _gather_sc(ta0_hbm, tb0_hbm, ta1_hbm, tb1_hbm, row_hbm, col_hbm, sr_out,
               rowv, colv, bufa, bufb, sem):
    c = lax.axis_index("c")
    s = lax.axis_index("s")
    base = (s * 2 + c) * EW

    def run(ta_hbm, tb_hbm):
        def chunk(g, _):
            eb = pl.multiple_of(base + g * GC, 8)
            pltpu.sync_copy(row_hbm.at[pl.ds(eb, GC)], rowv)
            pltpu.sync_copy(col_hbm.at[pl.ds(eb, GC)], colv)
            h1 = pltpu.async_copy(ta_hbm.at[rowv], bufa, sem)
            h2 = pltpu.async_copy(tb_hbm.at[colv], bufb, sem)
            h1.wait()
            h2.wait()

            def vrow(i, _):
                r = i * 4
                for rr in range(4):
                    for j in range(2):
                        cs = pl.ds(j * 32, 32)
                        bufa[r + rr, cs] = bufa[r + rr, cs] + bufb[r + rr, cs]
                    for j in range(2, 4):
                        cs = pl.ds(j * 32, 32)
                        bufa[r + rr, cs] = bufa[r + rr, cs] - bufb[r + rr, cs]
                return 0

            lax.fori_loop(0, GC // 4, vrow, 0)
            pltpu.sync_copy(bufa, sr_out.at[pl.ds(eb, GC)])
            return 0

        lax.fori_loop(0, GCH, chunk, 0)

    @pl.when(c == 0)
    def _():
        run(ta0_hbm, tb0_hbm)

    @pl.when(c == 1)
    def _():
        run(ta1_hbm, tb1_hbm)


@functools.partial(
    pl.kernel,
    mesh=_mesh,
    compiler_params=pltpu.CompilerParams(use_tc_tiling_on_sc=False),
    out_type=jax.ShapeDtypeStruct((N, W), _bf16),
    scratch_types=[
        pltpu.VMEM((SSUB,), jnp.int32),
        pltpu.VMEM((SSUB,), jnp.int32),
        pltpu.VMEM((SSUB, W), _bf16),
        pltpu.VMEM_SHARED((RSH, W), _bf16),
    ],
)
def _scatter_sc(mc_hbm, col_hbm, acc_out, colv, idxv, databuf, acc_sh):
    c = lax.axis_index("c")
    s = lax.axis_index("s")

    for p in range(2):
        nbase = (2 * p + c) * QTR

        def zrow(i, _):
            for j in range(W // 32):
                databuf[i, pl.ds(j * 32, 32)] = jnp.zeros((32,), _bf16)
            return 0

        lax.fori_loop(0, SSUB, zrow, 0)
        r0 = pl.multiple_of(s * (RSH // 16), 8)
        pltpu.sync_copy(databuf, acc_sh.at[pl.ds(r0, SSUB)])
        pltpu.sync_copy(databuf.at[pl.ds(0, 272)], acc_sh.at[pl.ds(r0 + SSUB, 272)])
        plsc.subcore_barrier()

        def chunk(g, _):
            eb = pl.multiple_of(s * ET + g * SSUB, 8)
            pltpu.sync_copy(col_hbm.at[pl.ds(eb, SSUB)], colv)

            def ibody(i, _):
                v = colv[pl.ds(i * 16, 16)]
                l = v - nbase
                ok = (l >= 0) & (l < QTR)
                idxv[pl.ds(i * 16, 16)] = jnp.where(ok, l, QTR)
                return 0

            lax.fori_loop(0, SSUB // 16, ibody, 0)
            pltpu.sync_copy(mc_hbm.at[pl.ds(eb, SSUB)], databuf)
            pltpu.sync_copy(databuf, acc_sh.at[idxv], add=True)
            return 0

        lax.fori_loop(0, SCH, chunk, 0)
        plsc.subcore_barrier()

        ob = s * 780
        og = nbase + s * 780
        pltpu.sync_copy(acc_sh.at[pl.ds(ob, 780)], acc_out.at[pl.ds(og, 780)])

        @pl.when(s == 0)
        def _():
            pltpu.sync_copy(acc_sh.at[pl.ds(12480, 20)], acc_out.at[pl.ds(nbase + 12480, 20)])

        plsc.subcore_barrier()


def _silu(v):
    return v * jax.nn.sigmoid(v)


def _pre_body(h_ref, x_ref, w1r_ref, w1c_ref, ta0_ref, tb0_ref, ta1_ref, tb1_ref):
    h = h_ref[...]
    x16 = x_ref[...]
    z = jnp.zeros((h.shape[0], W - D - 16), jnp.float32)
    ta = jnp.concatenate(
        [jnp.dot(h, w1r_ref[...], preferred_element_type=jnp.float32), x16, z],
        axis=1).astype(_bf16)
    tb = jnp.concatenate(
        [jnp.dot(h, w1c_ref[...], preferred_element_type=jnp.float32), x16, z],
        axis=1).astype(_bf16)
    ta0_ref[...] = ta
    tb0_ref[...] = tb
    ta1_ref[...] = ta
    tb1_ref[...] = tb


def _edge_body(sr_ref, w1d_ref, be1_ref, we2_ref, be2_ref,
               wc1_ref, bc1_ref, wc2_ref, mc_ref):
    sr = sr_ref[...].astype(jnp.float32)
    rel = sr[:, D:D + 16]
    dist = jnp.sqrt(jnp.sum(rel * rel, axis=-1, keepdims=True))
    pre = sr[:, :D] + dist * w1d_ref[...] + be1_ref[...]
    t = _silu(pre)
    msg = jnp.dot(t, we2_ref[...], preferred_element_type=jnp.float32) + be2_ref[...]
    cpre = jnp.dot(msg, wc1_ref[...], preferred_element_type=jnp.float32) + bc1_ref[...]
    cw = jnp.sum(_silu(cpre) * wc2_ref[...], axis=-1, keepdims=True)
    z = jnp.zeros((msg.shape[0], W - D - 16), jnp.float32)
    mc_ref[...] = jnp.concatenate([msg, rel * cw, z], axis=1).astype(_bf16)


def _node_body(h_ref, acc_ref, x_ref, wn1a_ref, wn1b_ref, bn1_ref,
               wn2_ref, bn2_ref, hn_ref, xn_ref):
    h = h_ref[...]
    acc = acc_ref[...].astype(jnp.float32)
    pre = (jnp.dot(h, wn1a_ref[...], preferred_element_type=jnp.float32)
           + jnp.dot(acc[:, :D], wn1b_ref[...], preferred_element_type=jnp.float32)
           + bn1_ref[...])
    hn_ref[...] = h + jnp.dot(_silu(pre), wn2_ref[...],
                              preferred_element_type=jnp.float32) + bn2_ref[...]
    xn_ref[...] = x_ref[...] + acc[:, D:D + 16]


def _full(shape):
    return pl.BlockSpec(shape, lambda i: (0, 0))


def kernel(h, x, edge_index, We1, be1, We2, be2, Wn1, bn1, Wn2, bn2, Wc1, bc1, Wc2):
    f32 = jnp.float32
    row = edge_index[0]
    col = edge_index[1]
    pad = jnp.full((EPAD - E,), N, dtype=jnp.int32)
    row_p = jnp.concatenate([row, pad])
    col_p = jnp.concatenate([col, pad])

    h_pad = jnp.zeros((NPAD, D), f32).at[:N].set(h)
    x_pad = jnp.zeros((NPAD, 16), f32).at[:N, :3].set(x)
    x16 = x_pad[:N]

    bn = 2048
    ta0, tb0, ta1, tb1 = pl.pallas_call(
        _pre_body,
        grid=(NPAD // bn,),
        in_specs=[pl.BlockSpec((bn, D), lambda i: (i, 0)),
                  pl.BlockSpec((bn, 16), lambda i: (i, 0)),
                  _full((D, D)), _full((D, D))],
        out_specs=[pl.BlockSpec((bn, W), lambda i: (i, 0))] * 4,
        out_shape=[jax.ShapeDtypeStruct((NPAD, W), _bf16)] * 4,
    )(h_pad, x_pad, We1[:D], We1[D:2 * D])

    sr = _gather_sc(ta0, tb0, ta1, tb1, row_p, col_p)

    be = 2048
    w1d = We1[2 * D:2 * D + 1]
    mc = pl.pallas_call(
        _edge_body,
        grid=(EPAD // be,),
        in_specs=[pl.BlockSpec((be, W), lambda i: (i, 0)),
                  _full((1, D)), _full((1, D)), _full((D, D)), _full((1, D)),
                  _full((D, D)), _full((1, D)), _full((1, D))],
        out_specs=pl.BlockSpec((be, W), lambda i: (i, 0)),
        out_shape=jax.ShapeDtypeStruct((EPAD, W), _bf16),
    )(sr, w1d, be1.reshape(1, D), We2, be2.reshape(1, D),
      Wc1, bc1.reshape(1, D), Wc2.reshape(1, D))

    acc = _scatter_sc(mc, col_p)

    bnn = 2000
    h_new, x_new16 = pl.pallas_call(
        _node_body,
        grid=(N // bnn,),
        in_specs=[pl.BlockSpec((bnn, D), lambda i: (i, 0)),
                  pl.BlockSpec((bnn, W), lambda i: (i, 0)),
                  pl.BlockSpec((bnn, 16), lambda i: (i, 0)),
                  _full((D, D)), _full((D, D)), _full((1, D)),
                  _full((D, D)), _full((1, D))],
        out_specs=[pl.BlockSpec((bnn, D), lambda i: (i, 0)),
                   pl.BlockSpec((bnn, 16), lambda i: (i, 0))],
        out_shape=[jax.ShapeDtypeStruct((N, D), f32),
                   jax.ShapeDtypeStruct((N, 16), f32)],
    )(h, acc, x16, Wn1[:D], Wn1[D:], bn1.reshape(1, D), Wn2, bn2.reshape(1, D))

    return h_new, x_new16[:, :3]


# speedup vs baseline: 2.7797x; 1.1596x over previous
import functools

import jax
import jax.numpy as jnp
from jax import lax
from jax.experimental import pallas as pl
from jax.experimental.pallas import tpu as pltpu
from jax.experimental.pallas import tpu_sc as plsc

N = 50000
E = 800000
D = 64
W = 128

NPAD = 51200
EPAD = 819200
NW = 32
EW = EPAD // NW
GC = 512
GCH = EW // GC
ET = EPAD // 16

_mesh = plsc.VectorSubcoreMesh(core_axis_name="c", subcore_axis_name="s")
_bf16 = jnp.bfloat16


@functools.partial(
    pl.kernel,
    mesh=_mesh,
    compiler_params=pltpu.CompilerParams(use_tc_tiling_on_sc=False),
    out_type=jax.ShapeDtypeStruct((EPAD, W), _bf16),
    scratch_types=[
        pltpu.VMEM((GC,), jnp.int32),
        pltpu.VMEM((GC,), jnp.int32),
        pltpu.VMEM((GC, W), _bf16),
        pltpu.VMEM((GC, W), _bf16),
        pltpu.SemaphoreType.DMA,
    ],
)
def _gather_sc(ta0_hbm, tb0_hbm, ta1_hbm, tb1_hbm, row_hbm, col_hbm, sr_out,
               rowv, colv, bufa, bufb, sem):
    c = lax.axis_index("c")
    s = lax.axis_index("s")
    base = (s * 2 + c) * EW

    def run(ta_hbm, tb_hbm):
        def chunk(g, _):
            eb = pl.multiple_of(base + g * GC, 8)
            pltpu.sync_copy(row_hbm.at[pl.ds(eb, GC)], rowv)
            pltpu.sync_copy(col_hbm.at[pl.ds(eb, GC)], colv)
            h1 = pltpu.async_copy(ta_hbm.at[rowv], bufa, sem)
            h2 = pltpu.async_copy(tb_hbm.at[colv], bufb, sem)
            h1.wait()
            h2.wait()

            def vrow(i, _):
                r = i * 4
                for rr in range(4):
                    for j in range(2):
                        cs = pl.ds(j * 32, 32)
                        bufa[r + rr, cs] = bufa[r + rr, cs] + bufb[r + rr, cs]
                    for j in range(2, 4):
                        cs = pl.ds(j * 32, 32)
                        bufa[r + rr, cs] = bufa[r + rr, cs] - bufb[r + rr, cs]
                return 0

            lax.fori_loop(0, GC // 4, vrow, 0)
            pltpu.sync_copy(bufa, sr_out.at[pl.ds(eb, GC)])
            return 0

        lax.fori_loop(0, GCH, chunk, 0)

    @pl.when(c == 0)
    def _():
        run(ta0_hbm, tb0_hbm)

    @pl.when(c == 1)
    def _():
        run(ta1_hbm, tb1_hbm)


HALF = 25000
RSH = 25088
SSUB = 320
SCH = ET // SSUB


@functools.partial(
    pl.kernel,
    mesh=_mesh,
    compiler_params=pltpu.CompilerParams(use_tc_tiling_on_sc=False),
    out_type=jax.ShapeDtypeStruct((N, W), _bf16),
    scratch_types=[
        pltpu.VMEM((SSUB,), jnp.int32),
        pltpu.VMEM((SSUB,), jnp.int32),
        pltpu.VMEM((SSUB, W), _bf16),
        pltpu.VMEM_SHARED((RSH, W), _bf16),
    ],
)
def _scatter_sc(mc_hbm, col_hbm, acc_out, colv, idxv, databuf, acc_sh):
    c = lax.axis_index("c")
    s = lax.axis_index("s")
    nbase = c * HALF

    def zrow(i, _):
        for j in range(W // 32):
            databuf[i, pl.ds(j * 32, 32)] = jnp.zeros((32,), _bf16)
        return 0

    lax.fori_loop(0, SSUB, zrow, 0)
    for q in range(4):
        r0 = pl.multiple_of(s * (RSH // 16) + q * SSUB, 8)
        pltpu.sync_copy(databuf, acc_sh.at[pl.ds(r0, SSUB)])
    r0 = pl.multiple_of(s * (RSH // 16) + 4 * SSUB, 8)
    pltpu.sync_copy(databuf.at[pl.ds(0, 288)], acc_sh.at[pl.ds(r0, 288)])
    plsc.subcore_barrier()

    def chunk(g, _):
        eb = pl.multiple_of(s * ET + g * SSUB, 8)
        pltpu.sync_copy(col_hbm.at[pl.ds(eb, SSUB)], colv)

        def ibody(i, _):
            v = colv[pl.ds(i * 16, 16)]
            l = v - nbase
            ok = (l >= 0) & (l < HALF)
            idxv[pl.ds(i * 16, 16)] = jnp.where(ok, l, HALF)
            return 0

        lax.fori_loop(0, SSUB // 16, ibody, 0)
        pltpu.sync_copy(mc_hbm.at[pl.ds(eb, SSUB)], databuf)
        pltpu.sync_copy(databuf, acc_sh.at[idxv], add=True)
        return 0

    lax.fori_loop(0, SCH, chunk, 0)
    plsc.subcore_barrier()

    ob = s * 1560
    og = nbase + s * 1560
    pltpu.sync_copy(acc_sh.at[pl.ds(ob, 1560)], acc_out.at[pl.ds(og, 1560)])

    @pl.when(s == 0)
    def _():
        pltpu.sync_copy(acc_sh.at[pl.ds(24960, 40)], acc_out.at[pl.ds(nbase + 24960, 40)])


def _silu(v):
    return v * jax.nn.sigmoid(v)


def _pre_body(h_ref, x_ref, w1r_ref, w1c_ref, ta0_ref, tb0_ref, ta1_ref, tb1_ref):
    h = h_ref[...]
    x16 = x_ref[...]
    z = jnp.zeros((h.shape[0], W - D - 16), jnp.float32)
    ta = jnp.concatenate(
        [jnp.dot(h, w1r_ref[...], preferred_element_type=jnp.float32), x16, z],
        axis=1).astype(_bf16)
    tb = jnp.concatenate(
        [jnp.dot(h, w1c_ref[...], preferred_element_type=jnp.float32), x16, z],
        axis=1).astype(_bf16)
    ta0_ref[...] = ta
    tb0_ref[...] = tb
    ta1_ref[...] = ta
    tb1_ref[...] = tb


def _edge_body(sr_ref, w1d_ref, be1_ref, we2_ref, be2_ref,
               wc1_ref, bc1_ref, wc2_ref, mc_ref):
    sr = sr_ref[...].astype(jnp.float32)
    rel = sr[:, D:D + 16]
    dist = jnp.sqrt(jnp.sum(rel * rel, axis=-1, keepdims=True))
    pre = sr[:, :D] + dist * w1d_ref[...] + be1_ref[...]
    t = _silu(pre)
    msg = jnp.dot(t, we2_ref[...], preferred_element_type=jnp.float32) + be2_ref[...]
    cpre = jnp.dot(msg, wc1_ref[...], preferred_element_type=jnp.float32) + bc1_ref[...]
    cw = jnp.sum(_silu(cpre) * wc2_ref[...], axis=-1, keepdims=True)
    z = jnp.zeros((msg.shape[0], W - D - 16), jnp.float32)
    mc_ref[...] = jnp.concatenate([msg, rel * cw, z], axis=1).astype(_bf16)


def _node_body(h_ref, acc_ref, x_ref, wn1a_ref, wn1b_ref, bn1_ref,
               wn2_ref, bn2_ref, hn_ref, xn_ref):
    h = h_ref[...]
    acc = acc_ref[...].astype(jnp.float32)
    pre = (jnp.dot(h, wn1a_ref[...], preferred_element_type=jnp.float32)
           + jnp.dot(acc[:, :D], wn1b_ref[...], preferred_element_type=jnp.float32)
           + bn1_ref[...])
    hn_ref[...] = h + jnp.dot(_silu(pre), wn2_ref[...],
                              preferred_element_type=jnp.float32) + bn2_ref[...]
    xn_ref[...] = x_ref[...] + acc[:, D:D + 16]


def _full(shape):
    return pl.BlockSpec(shape, lambda i: (0, 0))


def kernel(h, x, edge_index, We1, be1, We2, be2, Wn1, bn1, Wn2, bn2, Wc1, bc1, Wc2):
    f32 = jnp.float32
    row = edge_index[0]
    col = edge_index[1]
    pad = jnp.full((EPAD - E,), N, dtype=jnp.int32)
    row_p = jnp.concatenate([row, pad])
    col_p = jnp.concatenate([col, pad])

    h_pad = jnp.zeros((NPAD, D), f32).at[:N].set(h)
    x_pad = jnp.zeros((NPAD, 16), f32).at[:N, :3].set(x)
    x16 = x_pad[:N]

    bn = 2048
    ta0, tb0, ta1, tb1 = pl.pallas_call(
        _pre_body,
        grid=(NPAD // bn,),
        in_specs=[pl.BlockSpec((bn, D), lambda i: (i, 0)),
                  pl.BlockSpec((bn, 16), lambda i: (i, 0)),
                  _full((D, D)), _full((D, D))],
        out_specs=[pl.BlockSpec((bn, W), lambda i: (i, 0))] * 4,
        out_shape=[jax.ShapeDtypeStruct((NPAD, W), _bf16)] * 4,
    )(h_pad, x_pad, We1[:D], We1[D:2 * D])

    sr = _gather_sc(ta0, tb0, ta1, tb1, row_p, col_p)

    be = 2048
    w1d = We1[2 * D:2 * D + 1]
    mc = pl.pallas_call(
        _edge_body,
        grid=(EPAD // be,),
        in_specs=[pl.BlockSpec((be, W), lambda i: (i, 0)),
                  _full((1, D)), _full((1, D)), _full((D, D)), _full((1, D)),
                  _full((D, D)), _full((1, D)), _full((1, D))],
        out_specs=pl.BlockSpec((be, W), lambda i: (i, 0)),
        out_shape=jax.ShapeDtypeStruct((EPAD, W), _bf16),
    )(sr, w1d, be1.reshape(1, D), We2, be2.reshape(1, D),
      Wc1, bc1.reshape(1, D), Wc2.reshape(1, D))

    acc = _scatter_sc(mc, col_p)

    bnn = 2000
    h_new, x_new16 = pl.pallas_call(
        _node_body,
        grid=(N // bnn,),
        in_specs=[pl.BlockSpec((bnn, D), lambda i: (i, 0)),
                  pl.BlockSpec((bnn, W), lambda i: (i, 0)),
                  pl.BlockSpec((bnn, 16), lambda i: (i, 0)),
                  _full((D, D)), _full((D, D)), _full((1, D)),
                  _full((D, D)), _full((1, D))],
        out_specs=[pl.BlockSpec((bnn, D), lambda i: (i, 0)),
                   pl.BlockSpec((bnn, 16), lambda i: (i, 0))],
        out_shape=[jax.ShapeDtypeStruct((N, D), f32),
                   jax.ShapeDtypeStruct((N, 16), f32)],
    )(h, acc, x16, Wn1[:D], Wn1[D:], bn1.reshape(1, D), Wn2, bn2.reshape(1, D))

    return h_new, x_new16[:, :3]


# speedup vs baseline: 2.9293x; 1.0538x over previous
import functools

import jax
import jax.numpy as jnp
from jax import lax
from jax.experimental import pallas as pl
from jax.experimental.pallas import tpu as pltpu
from jax.experimental.pallas import tpu_sc as plsc

N = 50000
E = 800000
D = 64
W = 128

NPAD = 51200
EPAD = 819200
NW = 32
EHALF = EPAD // 2
EW = EHALF // NW
GC = 512
GCH = EW // GC
ET = EHALF // 16

_mesh = plsc.VectorSubcoreMesh(core_axis_name="c", subcore_axis_name="s")
_bf16 = jnp.bfloat16


@functools.partial(
    pl.kernel,
    mesh=_mesh,
    compiler_params=pltpu.CompilerParams(use_tc_tiling_on_sc=False),
    out_type=jax.ShapeDtypeStruct((EHALF, W), _bf16),
    scratch_types=[
        pltpu.VMEM((GC,), jnp.int32),
        pltpu.VMEM((GC,), jnp.int32),
        pltpu.VMEM((GC, W), _bf16),
        pltpu.VMEM((GC, W), _bf16),
        pltpu.SemaphoreType.DMA,
    ],
)
def _gather_sc(ta0_hbm, tb0_hbm, ta1_hbm, tb1_hbm, row_hbm, col_hbm, sr_out,
               rowv, colv, bufa, bufb, sem):
    c = lax.axis_index("c")
    s = lax.axis_index("s")
    base = (s * 2 + c) * EW

    def run(ta_hbm, tb_hbm):
        def chunk(g, _):
            eb = pl.multiple_of(base + g * GC, 8)
            pltpu.sync_copy(row_hbm.at[pl.ds(eb, GC)], rowv)
            pltpu.sync_copy(col_hbm.at[pl.ds(eb, GC)], colv)
            h1 = pltpu.async_copy(ta_hbm.at[rowv], bufa, sem)
            h2 = pltpu.async_copy(tb_hbm.at[colv], bufb, sem)
            h1.wait()
            h2.wait()

            def vrow(i, _):
                r = i * 4
                for rr in range(4):
                    for j in range(2):
                        cs = pl.ds(j * 32, 32)
                        bufa[r + rr, cs] = bufa[r + rr, cs] + bufb[r + rr, cs]
                    for j in range(2, 4):
                        cs = pl.ds(j * 32, 32)
                        bufa[r + rr, cs] = bufa[r + rr, cs] - bufb[r + rr, cs]
                return 0

            lax.fori_loop(0, GC // 4, vrow, 0)
            pltpu.sync_copy(bufa, sr_out.at[pl.ds(eb, GC)])
            return 0

        lax.fori_loop(0, GCH, chunk, 0)

    @pl.when(c == 0)
    def _():
        run(ta0_hbm, tb0_hbm)

    @pl.when(c == 1)
    def _():
        run(ta1_hbm, tb1_hbm)


HALF = 25000
RSH = 25088
SSUB = 320
SCH = ET // SSUB


@functools.partial(
    pl.kernel,
    mesh=_mesh,
    compiler_params=pltpu.CompilerParams(use_tc_tiling_on_sc=False),
    out_type=jax.ShapeDtypeStruct((N, W), _bf16),
    scratch_types=[
        pltpu.VMEM((SSUB,), jnp.int32),
        pltpu.VMEM((SSUB,), jnp.int32),
        pltpu.VMEM((SSUB, W), _bf16),
        pltpu.VMEM_SHARED((RSH, W), _bf16),
    ],
)
def _scatter_sc(mc_hbm, col_hbm, acc_out, colv, idxv, databuf, acc_sh):
    c = lax.axis_index("c")
    s = lax.axis_index("s")
    nbase = c * HALF

    def zrow(i, _):
        for j in range(W // 32):
            databuf[i, pl.ds(j * 32, 32)] = jnp.zeros((32,), _bf16)
        return 0

    lax.fori_loop(0, SSUB, zrow, 0)
    for q in range(4):
        r0 = pl.multiple_of(s * (RSH // 16) + q * SSUB, 8)
        pltpu.sync_copy(databuf, acc_sh.at[pl.ds(r0, SSUB)])
    r0 = pl.multiple_of(s * (RSH // 16) + 4 * SSUB, 8)
    pltpu.sync_copy(databuf.at[pl.ds(0, 288)], acc_sh.at[pl.ds(r0, 288)])
    plsc.subcore_barrier()

    def chunk(g, _):
        eb = pl.multiple_of(s * ET + g * SSUB, 8)
        pltpu.sync_copy(col_hbm.at[pl.ds(eb, SSUB)], colv)

        def ibody(i, _):
            v = colv[pl.ds(i * 16, 16)]
            l = v - nbase
            ok = (l >= 0) & (l < HALF)
            idxv[pl.ds(i * 16, 16)] = jnp.where(ok, l, HALF)
            return 0

        lax.fori_loop(0, SSUB // 16, ibody, 0)
        pltpu.sync_copy(mc_hbm.at[pl.ds(eb, SSUB)], databuf)
        pltpu.sync_copy(databuf, acc_sh.at[idxv], add=True)
        return 0

    lax.fori_loop(0, SCH, chunk, 0)
    plsc.subcore_barrier()

    ob = s * 1560
    og = nbase + s * 1560
    pltpu.sync_copy(acc_sh.at[pl.ds(ob, 1560)], acc_out.at[pl.ds(og, 1560)])

    @pl.when(s == 0)
    def _():
        pltpu.sync_copy(acc_sh.at[pl.ds(24960, 40)], acc_out.at[pl.ds(nbase + 24960, 40)])


def _silu(v):
    return v * jax.nn.sigmoid(v)


def _pre_body(h_ref, x_ref, w1r_ref, w1c_ref, ta0_ref, tb0_ref, ta1_ref, tb1_ref):
    h = h_ref[...]
    x16 = x_ref[...]
    z = jnp.zeros((h.shape[0], W - D - 16), jnp.float32)
    ta = jnp.concatenate(
        [jnp.dot(h, w1r_ref[...], preferred_element_type=jnp.float32), x16, z],
        axis=1).astype(_bf16)
    tb = jnp.concatenate(
        [jnp.dot(h, w1c_ref[...], preferred_element_type=jnp.float32), x16, z],
        axis=1).astype(_bf16)
    ta0_ref[...] = ta
    tb0_ref[...] = tb
    ta1_ref[...] = ta
    tb1_ref[...] = tb


def _edge_body(sr_ref, w1d_ref, be1_ref, we2_ref, be2_ref,
               wc1_ref, bc1_ref, wc2_ref, mc_ref):
    sr = sr_ref[...].astype(jnp.float32)
    rel = sr[:, D:D + 16]
    dist = jnp.sqrt(jnp.sum(rel * rel, axis=-1, keepdims=True))
    pre = sr[:, :D] + dist * w1d_ref[...] + be1_ref[...]
    t = _silu(pre)
    msg = jnp.dot(t, we2_ref[...], preferred_element_type=jnp.float32) + be2_ref[...]
    cpre = jnp.dot(msg, wc1_ref[...], preferred_element_type=jnp.float32) + bc1_ref[...]
    cw = jnp.sum(_silu(cpre) * wc2_ref[...], axis=-1, keepdims=True)
    z = jnp.zeros((msg.shape[0], W - D - 16), jnp.float32)
    mc_ref[...] = jnp.concatenate([msg, rel * cw, z], axis=1).astype(_bf16)


def _node_body(h_ref, acc1_ref, acc2_ref, x_ref, wn1a_ref, wn1b_ref, bn1_ref,
               wn2_ref, bn2_ref, hn_ref, xn_ref):
    h = h_ref[...]
    acc = acc1_ref[...].astype(jnp.float32) + acc2_ref[...].astype(jnp.float32)
    pre = (jnp.dot(h, wn1a_ref[...], preferred_element_type=jnp.float32)
           + jnp.dot(acc[:, :D], wn1b_ref[...], preferred_element_type=jnp.float32)
           + bn1_ref[...])
    hn_ref[...] = h + jnp.dot(_silu(pre), wn2_ref[...],
                              preferred_element_type=jnp.float32) + bn2_ref[...]
    xn_ref[...] = x_ref[...] + acc[:, D:D + 16]


def _full(shape):
    return pl.BlockSpec(shape, lambda i: (0, 0))


def kernel(h, x, edge_index, We1, be1, We2, be2, Wn1, bn1, Wn2, bn2, Wc1, bc1, Wc2):
    f32 = jnp.float32
    row = edge_index[0]
    col = edge_index[1]
    pad = jnp.full((EPAD - E,), N, dtype=jnp.int32)
    row_p = jnp.concatenate([row, pad])
    col_p = jnp.concatenate([col, pad])

    h_pad = jnp.zeros((NPAD, D), f32).at[:N].set(h)
    x_pad = jnp.zeros((NPAD, 16), f32).at[:N, :3].set(x)
    x16 = x_pad[:N]

    bn = 2048
    ta0, tb0, ta1, tb1 = pl.pallas_call(
        _pre_body,
        grid=(NPAD // bn,),
        in_specs=[pl.BlockSpec((bn, D), lambda i: (i, 0)),
                  pl.BlockSpec((bn, 16), lambda i: (i, 0)),
                  _full((D, D)), _full((D, D))],
        out_specs=[pl.BlockSpec((bn, W), lambda i: (i, 0))] * 4,
        out_shape=[jax.ShapeDtypeStruct((NPAD, W), _bf16)] * 4,
    )(h_pad, x_pad, We1[:D], We1[D:2 * D])

    be = 2048
    w1d = We1[2 * D:2 * D + 1]

    def edge_mlp(sr):
        return pl.pallas_call(
            _edge_body,
            grid=(EHALF // be,),
            in_specs=[pl.BlockSpec((be, W), lambda i: (i, 0)),
                      _full((1, D)), _full((1, D)), _full((D, D)), _full((1, D)),
                      _full((D, D)), _full((1, D)), _full((1, D))],
            out_specs=pl.BlockSpec((be, W), lambda i: (i, 0)),
            out_shape=jax.ShapeDtypeStruct((EHALF, W), _bf16),
        )(sr, w1d, be1.reshape(1, D), We2, be2.reshape(1, D),
          Wc1, bc1.reshape(1, D), Wc2.reshape(1, D))

    row1, row2 = row_p[:EHALF], row_p[EHALF:]
    col1, col2 = col_p[:EHALF], col_p[EHALF:]
    sr1 = _gather_sc(ta0, tb0, ta1, tb1, row1, col1)
    sr2 = _gather_sc(ta0, tb0, ta1, tb1, row2, col2)
    mc1 = edge_mlp(sr1)
    mc2 = edge_mlp(sr2)
    acc1 = _scatter_sc(mc1, col1)
    acc2 = _scatter_sc(mc2, col2)

    bnn = 2000
    h_new, x_new16 = pl.pallas_call(
        _node_body,
        grid=(N // bnn,),
        in_specs=[pl.BlockSpec((bnn, D), lambda i: (i, 0)),
                  pl.BlockSpec((bnn, W), lambda i: (i, 0)),
                  pl.BlockSpec((bnn, W), lambda i: (i, 0)),
                  pl.BlockSpec((bnn, 16), lambda i: (i, 0)),
                  _full((D, D)), _full((D, D)), _full((1, D)),
                  _full((D, D)), _full((1, D))],
        out_specs=[pl.BlockSpec((bnn, D), lambda i: (i, 0)),
                   pl.BlockSpec((bnn, 16), lambda i: (i, 0))],
        out_shape=[jax.ShapeDtypeStruct((N, D), f32),
                   jax.ShapeDtypeStruct((N, 16), f32)],
    )(h, acc1, acc2, x16, Wn1[:D], Wn1[D:], bn1.reshape(1, D), Wn2, bn2.reshape(1, D))

    return h_new, x_new16[:, :3]


# speedup vs baseline: 3.2160x; 1.0979x over previous
import functools

import jax
import jax.numpy as jnp
from jax import lax
from jax.experimental import pallas as pl
from jax.experimental.pallas import tpu as pltpu
from jax.experimental.pallas import tpu_sc as plsc

N = 50000
E = 800000
D = 64
W = 128

NPAD = 51200
EPAD = 819200
NW = 32
EHALF = EPAD // 2
EW = EHALF // NW
GC = 256
GCH = EW // GC
ET = EHALF // 16

_mesh = plsc.VectorSubcoreMesh(core_axis_name="c", subcore_axis_name="s")
_bf16 = jnp.bfloat16


@functools.partial(
    pl.kernel,
    mesh=_mesh,
    compiler_params=pltpu.CompilerParams(use_tc_tiling_on_sc=False),
    out_type=jax.ShapeDtypeStruct((256,), jnp.int32),
    scratch_types=[pltpu.VMEM((128,), jnp.int32)],
)
def _warmup_sc(in_hbm, out_hbm, buf):
    c = lax.axis_index("c")
    s = lax.axis_index("s")

    @pl.when(s == 0)
    def _():
        off = pl.multiple_of(c * 128, 8)
        pltpu.sync_copy(in_hbm.at[pl.ds(off, 128)], buf)
        pltpu.sync_copy(buf, out_hbm.at[pl.ds(off, 128)])


@functools.partial(
    pl.kernel,
    mesh=_mesh,
    compiler_params=pltpu.CompilerParams(use_tc_tiling_on_sc=False),
    out_type=jax.ShapeDtypeStruct((EHALF, W), jnp.float32),
    scratch_types=[
        pltpu.VMEM((GC,), jnp.int32),
        pltpu.VMEM((GC,), jnp.int32),
        pltpu.VMEM((GC, W), jnp.float32),
        pltpu.VMEM((GC, W), jnp.float32),
        pltpu.SemaphoreType.DMA,
    ],
)
def _gather_sc(ta0_hbm, tb0_hbm, ta1_hbm, tb1_hbm, row_hbm, col_hbm, wu_hbm,
               sr_out, rowv, colv, bufa, bufb, sem):
    del wu_hbm
    c = lax.axis_index("c")
    s = lax.axis_index("s")
    base = (s * 2 + c) * EW

    def run(ta_hbm, tb_hbm):
        def chunk(g, _):
            eb = pl.multiple_of(base + g * GC, 8)
            pltpu.sync_copy(row_hbm.at[pl.ds(eb, GC)], rowv)
            pltpu.sync_copy(col_hbm.at[pl.ds(eb, GC)], colv)
            h1 = pltpu.async_copy(ta_hbm.at[rowv], bufa, sem)
            h2 = pltpu.async_copy(tb_hbm.at[colv], bufb, sem)
            h1.wait()
            h2.wait()

            def vrow(i, _):
                r = i * 4
                for rr in range(4):
                    for j in range(4):
                        cs = pl.ds(j * 16, 16)
                        bufa[r + rr, cs] = bufa[r + rr, cs] + bufb[r + rr, cs]
                    cs = pl.ds(64, 16)
                    bufa[r + rr, cs] = bufa[r + rr, cs] - bufb[r + rr, cs]
                return 0

            lax.fori_loop(0, GC // 4, vrow, 0)
            pltpu.sync_copy(bufa, sr_out.at[pl.ds(eb, GC)])
            return 0

        lax.fori_loop(0, GCH, chunk, 0)

    @pl.when(c == 0)
    def _():
        run(ta0_hbm, tb0_hbm)

    @pl.when(c == 1)
    def _():
        run(ta1_hbm, tb1_hbm)


HALF = 25000
RSH = 25088
SSUB = 320
SCH = ET // SSUB


@functools.partial(
    pl.kernel,
    mesh=_mesh,
    compiler_params=pltpu.CompilerParams(use_tc_tiling_on_sc=False),
    out_type=jax.ShapeDtypeStruct((N, W), _bf16),
    scratch_types=[
        pltpu.VMEM((SSUB,), jnp.int32),
        pltpu.VMEM((SSUB,), jnp.int32),
        pltpu.VMEM((SSUB, W), _bf16),
        pltpu.VMEM_SHARED((RSH, W), _bf16),
    ],
)
def _scatter_sc(mc_hbm, col_hbm, acc_out, colv, idxv, databuf, acc_sh):
    c = lax.axis_index("c")
    s = lax.axis_index("s")
    nbase = c * HALF

    def zrow(i, _):
        for j in range(W // 32):
            databuf[i, pl.ds(j * 32, 32)] = jnp.zeros((32,), _bf16)
        return 0

    lax.fori_loop(0, SSUB, zrow, 0)
    for q in range(4):
        r0 = pl.multiple_of(s * (RSH // 16) + q * SSUB, 8)
        pltpu.sync_copy(databuf, acc_sh.at[pl.ds(r0, SSUB)])
    r0 = pl.multiple_of(s * (RSH // 16) + 4 * SSUB, 8)
    pltpu.sync_copy(databuf.at[pl.ds(0, 288)], acc_sh.at[pl.ds(r0, 288)])
    plsc.subcore_barrier()

    def chunk(g, _):
        eb = pl.multiple_of(s * ET + g * SSUB, 8)
        pltpu.sync_copy(col_hbm.at[pl.ds(eb, SSUB)], colv)

        def ibody(i, _):
            v = colv[pl.ds(i * 16, 16)]
            l = v - nbase
            ok = (l >= 0) & (l < HALF)
            idxv[pl.ds(i * 16, 16)] = jnp.where(ok, l, HALF)
            return 0

        lax.fori_loop(0, SSUB // 16, ibody, 0)
        pltpu.sync_copy(mc_hbm.at[pl.ds(eb, SSUB)], databuf)
        pltpu.sync_copy(databuf, acc_sh.at[idxv], add=True)
        return 0

    lax.fori_loop(0, SCH, chunk, 0)
    plsc.subcore_barrier()

    ob = s * 1560
    og = nbase + s * 1560
    pltpu.sync_copy(acc_sh.at[pl.ds(ob, 1560)], acc_out.at[pl.ds(og, 1560)])

    @pl.when(s == 0)
    def _():
        pltpu.sync_copy(acc_sh.at[pl.ds(24960, 40)], acc_out.at[pl.ds(nbase + 24960, 40)])


def _silu(v):
    return v * jax.nn.sigmoid(v)


def _pre_body(h_ref, x_ref, w1r_ref, w1c_ref, ta0_ref, tb0_ref, ta1_ref, tb1_ref):
    h = h_ref[...]
    x16 = x_ref[...]
    z = jnp.zeros((h.shape[0], W - D - 16), jnp.float32)
    ta = jnp.concatenate(
        [jnp.dot(h, w1r_ref[...], preferred_element_type=jnp.float32), x16, z],
        axis=1)
    tb = jnp.concatenate(
        [jnp.dot(h, w1c_ref[...], preferred_element_type=jnp.float32), x16, z],
        axis=1)
    ta0_ref[...] = ta
    tb0_ref[...] = tb
    ta1_ref[...] = ta
    tb1_ref[...] = tb


def _edge_body(sr_ref, w1d_ref, be1_ref, we2_ref, be2_ref,
               wc1_ref, bc1_ref, wc2_ref, mc_ref):
    sr = sr_ref[...]
    rel = sr[:, D:D + 16]
    dist = jnp.sqrt(jnp.sum(rel * rel, axis=-1, keepdims=True))
    pre = sr[:, :D] + dist * w1d_ref[...] + be1_ref[...]
    t = _silu(pre)
    msg = jnp.dot(t, we2_ref[...], preferred_element_type=jnp.float32) + be2_ref[...]
    cpre = jnp.dot(msg, wc1_ref[...], preferred_element_type=jnp.float32) + bc1_ref[...]
    cw = jnp.sum(_silu(cpre) * wc2_ref[...], axis=-1, keepdims=True)
    z = jnp.zeros((msg.shape[0], W - D - 16), jnp.float32)
    mc_ref[...] = jnp.concatenate([msg, rel * cw, z], axis=1).astype(_bf16)


def _node_body(h_ref, acc1_ref, acc2_ref, x_ref, wn1a_ref, wn1b_ref, bn1_ref,
               wn2_ref, bn2_ref, hn_ref, xn_ref):
    h = h_ref[...]
    acc = acc1_ref[...].astype(jnp.float32) + acc2_ref[...].astype(jnp.float32)
    pre = (jnp.dot(h, wn1a_ref[...], preferred_element_type=jnp.float32)
           + jnp.dot(acc[:, :D], wn1b_ref[...], preferred_element_type=jnp.float32)
           + bn1_ref[...])
    hn_ref[...] = h + jnp.dot(_silu(pre), wn2_ref[...],
                              preferred_element_type=jnp.float32) + bn2_ref[...]
    xn_ref[...] = x_ref[...] + acc[:, D:D + 16]


def _full(shape):
    return pl.BlockSpec(shape, lambda i: (0, 0))


def kernel(h, x, edge_index, We1, be1, We2, be2, Wn1, bn1, Wn2, bn2, Wc1, bc1, Wc2):
    f32 = jnp.float32
    row = edge_index[0]
    col = edge_index[1]
    pad = jnp.full((EPAD - E,), N, dtype=jnp.int32)
    row_p = jnp.concatenate([row, pad])
    col_p = jnp.concatenate([col, pad])

    h_pad = jnp.zeros((NPAD, D), f32).at[:N].set(h)
    x_pad = jnp.zeros((NPAD, 16), f32).at[:N, :3].set(x)
    x16 = x_pad[:N]

    bn = 2048
    ta0, tb0, ta1, tb1 = pl.pallas_call(
        _pre_body,
        grid=(NPAD // bn,),
        in_specs=[pl.BlockSpec((bn, D), lambda i: (i, 0)),
                  pl.BlockSpec((bn, 16), lambda i: (i, 0)),
                  _full((D, D)), _full((D, D))],
        out_specs=[pl.BlockSpec((bn, W), lambda i: (i, 0))] * 4,
        out_shape=[jax.ShapeDtypeStruct((NPAD, W), f32)] * 4,
    )(h_pad, x_pad, We1[:D], We1[D:2 * D])

    be = 2048
    w1d = We1[2 * D:2 * D + 1]

    def edge_mlp(sr):
        return pl.pallas_call(
            _edge_body,
            grid=(EHALF // be,),
            in_specs=[pl.BlockSpec((be, W), lambda i: (i, 0)),
                      _full((1, D)), _full((1, D)), _full((D, D)), _full((1, D)),
                      _full((D, D)), _full((1, D)), _full((1, D))],
            out_specs=pl.BlockSpec((be, W), lambda i: (i, 0)),
            out_shape=jax.ShapeDtypeStruct((EHALF, W), _bf16),
        )(sr, w1d, be1.reshape(1, D), We2, be2.reshape(1, D),
          Wc1, bc1.reshape(1, D), Wc2.reshape(1, D))

    row1, row2 = row_p[:EHALF], row_p[EHALF:]
    col1, col2 = col_p[:EHALF], col_p[EHALF:]
    wu = _warmup_sc(col_p[:256])
    sr1 = _gather_sc(ta0, tb0, ta1, tb1, row1, col1, wu)
    sr2 = _gather_sc(ta0, tb0, ta1, tb1, row2, col2, wu)
    mc1 = edge_mlp(sr1)
    mc2 = edge_mlp(sr2)
    acc1 = _scatter_sc(mc1, col1)
    acc2 = _scatter_sc(mc2, col2)

    bnn = 2000
    h_new, x_new16 = pl.pallas_call(
        _node_body,
        grid=(N // bnn,),
        in_specs=[pl.BlockSpec((bnn, D), lambda i: (i, 0)),
                  pl.BlockSpec((bnn, W), lambda i: (i, 0)),
                  pl.BlockSpec((bnn, W), lambda i: (i, 0)),
                  pl.BlockSpec((bnn, 16), lambda i: (i, 0)),
                  _full((D, D)), _full((D, D)), _full((1, D)),
                  _full((D, D)), _full((1, D))],
        out_specs=[pl.BlockSpec((bnn, D), lambda i: (i, 0)),
                   pl.BlockSpec((bnn, 16), lambda i: (i, 0))],
        out_shape=[jax.ShapeDtypeStruct((N, D), f32),
                   jax.ShapeDtypeStruct((N, 16), f32)],
    )(h, acc1, acc2, x16, Wn1[:D], Wn1[D:], bn1.reshape(1, D), Wn2, bn2.reshape(1, D))

    return h_new, x_new16[:, :3]


# speedup vs baseline: 3.3227x; 1.0332x over previous
import functools

import jax
import jax.numpy as jnp
from jax import lax
from jax.experimental import pallas as pl
from jax.experimental.pallas import tpu as pltpu
from jax.experimental.pallas import tpu_sc as plsc

N = 50000
E = 800000
D = 64
W = 128

NPAD = 51200
EPAD = 819200
NW = 32
EHALF = EPAD // 2
EW = EHALF // NW
GC = 256
GCH = EW // GC
ET = EHALF // 16

_mesh = plsc.VectorSubcoreMesh(core_axis_name="c", subcore_axis_name="s")
_bf16 = jnp.bfloat16


@functools.partial(
    pl.kernel,
    mesh=_mesh,
    compiler_params=pltpu.CompilerParams(use_tc_tiling_on_sc=False),
    out_type=jax.ShapeDtypeStruct((256,), jnp.int32),
    scratch_types=[pltpu.VMEM((128,), jnp.int32)],
)
def _warmup_sc(in_hbm, out_hbm, buf):
    c = lax.axis_index("c")
    s = lax.axis_index("s")

    @pl.when(s == 0)
    def _():
        off = pl.multiple_of(c * 128, 8)
        pltpu.sync_copy(in_hbm.at[pl.ds(off, 128)], buf)
        pltpu.sync_copy(buf, out_hbm.at[pl.ds(off, 128)])


@functools.partial(
    pl.kernel,
    mesh=_mesh,
    compiler_params=pltpu.CompilerParams(use_tc_tiling_on_sc=False),
    out_type=jax.ShapeDtypeStruct((EHALF, W), jnp.float32),
    scratch_types=[
        pltpu.VMEM((GC,), jnp.int32),
        pltpu.VMEM((GC,), jnp.int32),
        pltpu.VMEM((GC, W), jnp.float32),
        pltpu.VMEM((GC, W), jnp.float32),
        pltpu.SemaphoreType.DMA,
    ],
)
def _gather_sc(ta0_hbm, tb0_hbm, ta1_hbm, tb1_hbm, row_hbm, col_hbm, wu_hbm,
               sr_out, rowv, colv, bufa, bufb, sem):
    del wu_hbm
    c = lax.axis_index("c")
    s = lax.axis_index("s")
    base = (s * 2 + c) * EW

    def run(ta_hbm, tb_hbm):
        def chunk(g, _):
            eb = pl.multiple_of(base + g * GC, 8)
            pltpu.sync_copy(row_hbm.at[pl.ds(eb, GC)], rowv)
            pltpu.sync_copy(col_hbm.at[pl.ds(eb, GC)], colv)
            h1 = pltpu.async_copy(ta_hbm.at[rowv], bufa, sem)
            h2 = pltpu.async_copy(tb_hbm.at[colv], bufb, sem)
            h1.wait()
            h2.wait()

            def vrow(i, _):
                r = i * 4
                for rr in range(4):
                    for j in range(4):
                        cs = pl.ds(j * 16, 16)
                        bufa[r + rr, cs] = bufa[r + rr, cs] + bufb[r + rr, cs]
                    cs = pl.ds(64, 16)
                    bufa[r + rr, cs] = bufa[r + rr, cs] - bufb[r + rr, cs]
                return 0

            lax.fori_loop(0, GC // 4, vrow, 0)
            pltpu.sync_copy(bufa, sr_out.at[pl.ds(eb, GC)])
            return 0

        lax.fori_loop(0, GCH, chunk, 0)

    @pl.when(c == 0)
    def _():
        run(ta0_hbm, tb0_hbm)

    @pl.when(c == 1)
    def _():
        run(ta1_hbm, tb1_hbm)


HALF = 25000
RSH = 25088
SSUB = 128
SCH = ET // SSUB


@functools.partial(
    pl.kernel,
    mesh=_mesh,
    compiler_params=pltpu.CompilerParams(use_tc_tiling_on_sc=False,
                                         needs_layout_passes=False),
    out_type=jax.ShapeDtypeStruct((N, W), _bf16),
    scratch_types=[
        pltpu.VMEM((SSUB,), jnp.int32),
        pltpu.VMEM((SSUB,), jnp.int32),
        pltpu.VMEM((SSUB, W), jnp.float32),
        pltpu.VMEM((SSUB, W), _bf16),
        pltpu.VMEM_SHARED((RSH, W), _bf16),
    ],
)
def _scatter_sc(mc_hbm, col_hbm, acc_out, colv, idxv, loadbuf, databuf, acc_sh):
    c = lax.axis_index("c")
    s = lax.axis_index("s")
    nbase = c * HALF

    def zrow(i, _):
        for j in range(W // 32):
            databuf[i, pl.ds(j * 32, 32)] = jnp.zeros((32,), _bf16)
        return 0

    lax.fori_loop(0, SSUB, zrow, 0)
    for q in range(12):
        r0 = pl.multiple_of(s * (RSH // 16) + q * SSUB, 8)
        pltpu.sync_copy(databuf, acc_sh.at[pl.ds(r0, SSUB)])
    r0 = pl.multiple_of(s * (RSH // 16) + 12 * SSUB, 8)
    pltpu.sync_copy(databuf.at[pl.ds(0, 32)], acc_sh.at[pl.ds(r0, 32)])
    plsc.subcore_barrier()

    def chunk(g, _):
        eb = pl.multiple_of(s * ET + g * SSUB, 8)
        pltpu.sync_copy(col_hbm.at[pl.ds(eb, SSUB)], colv)

        def ibody(i, _):
            v = colv[pl.ds(i * 16, 16)]
            l = v - nbase
            ok = (l >= 0) & (l < HALF)
            idxv[pl.ds(i * 16, 16)] = jnp.where(ok, l, HALF)
            return 0

        lax.fori_loop(0, SSUB // 16, ibody, 0)
        pltpu.sync_copy(mc_hbm.at[pl.ds(eb, SSUB)], loadbuf)

        def prow(r, _):
            for g in range(3):
                a = loadbuf[r, pl.ds(g * 32, 16)]
                b = loadbuf[r, pl.ds(g * 32 + 16, 16)]
                databuf[r, pl.ds(g * 32, 32)] = plsc.pack(
                    a, b, format=plsc.PackFormat.INTERLEAVED)
            return 0

        lax.fori_loop(0, SSUB, prow, 0)
        pltpu.sync_copy(databuf, acc_sh.at[idxv], add=True)
        return 0

    lax.fori_loop(0, SCH, chunk, 0)
    plsc.subcore_barrier()

    ob = s * 1560
    og = nbase + s * 1560
    pltpu.sync_copy(acc_sh.at[pl.ds(ob, 1560)], acc_out.at[pl.ds(og, 1560)])

    @pl.when(s == 0)
    def _():
        pltpu.sync_copy(acc_sh.at[pl.ds(24960, 40)], acc_out.at[pl.ds(nbase + 24960, 40)])


def _silu(v):
    return v * jax.nn.sigmoid(v)


def _pre_body(h_ref, x_ref, w1r_ref, w1c_ref, ta0_ref, tb0_ref, ta1_ref, tb1_ref):
    h = h_ref[...]
    x16 = x_ref[...]
    z = jnp.zeros((h.shape[0], W - D - 16), jnp.float32)
    ta = jnp.concatenate(
        [jnp.dot(h, w1r_ref[...], preferred_element_type=jnp.float32), x16, z],
        axis=1)
    tb = jnp.concatenate(
        [jnp.dot(h, w1c_ref[...], preferred_element_type=jnp.float32), x16, z],
        axis=1)
    ta0_ref[...] = ta
    tb0_ref[...] = tb
    ta1_ref[...] = ta
    tb1_ref[...] = tb


def _edge_body(sr_ref, w1d_ref, be1_ref, we2_ref, be2_ref,
               wc1_ref, bc1_ref, wc2_ref, mc_ref):
    sr = sr_ref[...]
    rel = sr[:, D:D + 16]
    dist = jnp.sqrt(jnp.sum(rel * rel, axis=-1, keepdims=True))
    pre = sr[:, :D] + dist * w1d_ref[...] + be1_ref[...]
    t = _silu(pre)
    msg = jnp.dot(t, we2_ref[...], preferred_element_type=jnp.float32) + be2_ref[...]
    cpre = jnp.dot(msg, wc1_ref[...], preferred_element_type=jnp.float32) + bc1_ref[...]
    cw = jnp.sum(_silu(cpre) * wc2_ref[...], axis=-1, keepdims=True)
    z = jnp.zeros((msg.shape[0], W - D - 16), jnp.float32)
    mc_ref[...] = jnp.concatenate([msg, rel * cw, z], axis=1)


_PSRC = []
for _g in range(4):
    for _i in range(16):
        _PSRC.extend([32 * _g + _i, 32 * _g + 16 + _i])
_XIDX = tuple(_PSRC.index(D + k) for k in range(16))


def _node_body(h_ref, acc1_ref, acc2_ref, x_ref, wn1a_ref, wn1bp_ref, bn1_ref,
               wn2_ref, bn2_ref, hn_ref, xn_ref):
    h = h_ref[...]
    acc = acc1_ref[...].astype(jnp.float32) + acc2_ref[...].astype(jnp.float32)
    pre = (jnp.dot(h, wn1a_ref[...], preferred_element_type=jnp.float32)
           + jnp.dot(acc[:, :D], wn1bp_ref[...], preferred_element_type=jnp.float32)
           + bn1_ref[...])
    hn_ref[...] = h + jnp.dot(_silu(pre), wn2_ref[...],
                              preferred_element_type=jnp.float32) + bn2_ref[...]
    xacc = jnp.concatenate([acc[:, j:j + 1] for j in _XIDX], axis=1)
    xn_ref[...] = x_ref[...] + xacc


def _full(shape):
    return pl.BlockSpec(shape, lambda i: (0, 0))


def kernel(h, x, edge_index, We1, be1, We2, be2, Wn1, bn1, Wn2, bn2, Wc1, bc1, Wc2):
    f32 = jnp.float32
    row = edge_index[0]
    col = edge_index[1]
    pad = jnp.full((EPAD - E,), N, dtype=jnp.int32)
    row_p = jnp.concatenate([row, pad])
    col_p = jnp.concatenate([col, pad])

    h_pad = jnp.zeros((NPAD, D), f32).at[:N].set(h)
    x_pad = jnp.zeros((NPAD, 16), f32).at[:N, :3].set(x)
    x16 = x_pad[:N]

    bn = 2048
    ta0, tb0, ta1, tb1 = pl.pallas_call(
        _pre_body,
        grid=(NPAD // bn,),
        in_specs=[pl.BlockSpec((bn, D), lambda i: (i, 0)),
                  pl.BlockSpec((bn, 16), lambda i: (i, 0)),
                  _full((D, D)), _full((D, D))],
        out_specs=[pl.BlockSpec((bn, W), lambda i: (i, 0))] * 4,
        out_shape=[jax.ShapeDtypeStruct((NPAD, W), f32)] * 4,
    )(h_pad, x_pad, We1[:D], We1[D:2 * D])

    be = 2048
    w1d = We1[2 * D:2 * D + 1]

    def edge_mlp(sr):
        return pl.pallas_call(
            _edge_body,
            grid=(EHALF // be,),
            in_specs=[pl.BlockSpec((be, W), lambda i: (i, 0)),
                      _full((1, D)), _full((1, D)), _full((D, D)), _full((1, D)),
                      _full((D, D)), _full((1, D)), _full((1, D))],
            out_specs=pl.BlockSpec((be, W), lambda i: (i, 0)),
            out_shape=jax.ShapeDtypeStruct((EHALF, W), jnp.float32),
        )(sr, w1d, be1.reshape(1, D), We2, be2.reshape(1, D),
          Wc1, bc1.reshape(1, D), Wc2.reshape(1, D))

    row1, row2 = row_p[:EHALF], row_p[EHALF:]
    col1, col2 = col_p[:EHALF], col_p[EHALF:]
    wu = _warmup_sc(col_p[:256])
    sr1 = _gather_sc(ta0, tb0, ta1, tb1, row1, col1, wu)
    sr2 = _gather_sc(ta0, tb0, ta1, tb1, row2, col2, wu)
    mc1 = edge_mlp(sr1)
    mc2 = edge_mlp(sr2)
    acc1 = _scatter_sc(mc1, col1)
    acc2 = _scatter_sc(mc2, col2)

    bnn = 2000
    _node_call = pl.pallas_call(
        _node_body,
        grid=(N // bnn,),
        in_specs=[pl.BlockSpec((bnn, D), lambda i: (i, 0)),
                  pl.BlockSpec((bnn, W), lambda i: (i, 0)),
                  pl.BlockSpec((bnn, W), lambda i: (i, 0)),
                  pl.BlockSpec((bnn, 16), lambda i: (i, 0)),
                  _full((D, D)), _full((D, D)), _full((1, D)),
                  _full((D, D)), _full((1, D))],
        out_specs=[pl.BlockSpec((bnn, D), lambda i: (i, 0)),
                   pl.BlockSpec((bnn, 16), lambda i: (i, 0))],
        out_shape=[jax.ShapeDtypeStruct((N, D), f32),
                   jax.ShapeDtypeStruct((N, 16), f32)],
    )
    wn1b_perm = Wn1[D:][jnp.array(_PSRC[:D]), :]
    h_new, x_new16 = _node_call(h, acc1, acc2, x16, Wn1[:D], wn1b_perm,
                                bn1.reshape(1, D), Wn2, bn2.reshape(1, D))

    return h_new, x_new16[:, :3]
